# Initial kernel scaffold; baseline (speedup 1.0000x reference)
import jax
import jax.numpy as jnp
from jax.experimental import pallas as pl


def kernel(x, edge_index, edge_weight, W1, b1, W2, b2, Wl, bl):
    raise NotImplementedError("write your pallas kernel here")


# speedup vs baseline: 3.2527x; 3.2527x over previous
import functools

import jax
import jax.numpy as jnp
from jax import lax
from jax.experimental import pallas as pl
from jax.experimental.pallas import tpu as pltpu
from jax.experimental.pallas import tpu_sc as plsc

NC = 2
NS = 16
NW = NC * NS
CH = 128


def _mm_body(x_ref, w_ref, o_ref):
    o_ref[...] = jnp.dot(x_ref[...], w_ref[...],
                         preferred_element_type=jnp.float32)


def _tc_matmul(x, w, br=2000):
    n, k = x.shape
    m = w.shape[1]
    return pl.pallas_call(
        _mm_body,
        grid=(n // br,),
        in_specs=[pl.BlockSpec((br, k), lambda i: (i, 0)),
                  pl.BlockSpec((k, m), lambda i: (0, 0))],
        out_specs=pl.BlockSpec((br, m), lambda i: (i, 0)),
        out_shape=jax.ShapeDtypeStruct((n, m), jnp.float32),
    )(x, w)


def _fused_body(p0_ref, p1_ref, b_ref, w_ref, bo_ref, o_ref):
    h = jnp.maximum(p0_ref[...] + p1_ref[...] + b_ref[...], 0.0)
    o_ref[...] = jnp.dot(h, w_ref[...],
                         preferred_element_type=jnp.float32) + bo_ref[...]


def _tc_fused(p0, p1, b, w, bo, br=2000):
    n, k = p0.shape
    m = w.shape[1]
    return pl.pallas_call(
        _fused_body,
        grid=(n // br,),
        in_specs=[pl.BlockSpec((br, k), lambda i: (i, 0)),
                  pl.BlockSpec((br, k), lambda i: (i, 0)),
                  pl.BlockSpec((1, k), lambda i: (0, 0)),
                  pl.BlockSpec((k, m), lambda i: (0, 0)),
                  pl.BlockSpec((1, m), lambda i: (0, 0))],
        out_specs=pl.BlockSpec((br, m), lambda i: (i, 0)),
        out_shape=jax.ShapeDtypeStruct((n, m), jnp.float32),
    )(p0, p1, b.reshape(1, k), w, bo.reshape(1, m))


def _sc_scatter(h, src, dst, ew):
    n, d = h.shape
    epw = src.shape[0] // NW
    nchunk = epw // CH
    npad = ((n + NS * CH - 1) // (NS * CH)) * (NS * CH)
    rpt = npad // NS
    mesh = plsc.VectorSubcoreMesh(core_axis_name="c", subcore_axis_name="s")

    @functools.partial(
        pl.kernel,
        out_type=jax.ShapeDtypeStruct((NC, npad, d), jnp.float32),
        mesh=mesh,
        scratch_types=[
            pltpu.VMEM((CH,), jnp.int32),
            pltpu.VMEM((CH,), jnp.int32),
            pltpu.VMEM((CH,), jnp.float32),
            pltpu.VMEM((CH, d), jnp.float32),
            pltpu.VMEM_SHARED((npad, d), jnp.float32),
            pltpu.SemaphoreType.DMA,
        ],
    )
    def body(h_hbm, src_hbm, dst_hbm, ew_hbm, out_hbm,
             src_v, dst_v, ew_v, rows_v, acc, sem):
        c = lax.axis_index("c")
        s = lax.axis_index("s")
        wid = s * NC + c

        def zrow(i, _):
            for j in range(d // 16):
                rows_v[i, pl.ds(j * 16, 16)] = jnp.zeros((16,), jnp.float32)
            return 0
        lax.fori_loop(0, CH, zrow, 0)
        for r in range(rpt // CH):
            pltpu.sync_copy(rows_v, acc.at[pl.ds(s * rpt + r * CH, CH)])
        plsc.subcore_barrier()

        base = wid * epw

        def chunk(kc, _):
            off = pl.multiple_of(base + kc * CH, CH)
            pltpu.sync_copy(src_hbm.at[pl.ds(off, CH)], src_v)
            pltpu.sync_copy(ew_hbm.at[pl.ds(off, CH)], ew_v)
            pltpu.sync_copy(dst_hbm.at[pl.ds(off, CH)], dst_v)
            pltpu.async_copy(h_hbm.at[src_v], rows_v, sem).wait()

            def grp(g, _):
                wv = ew_v[pl.ds(g * 16, 16)]
                for lane in range(16):
                    w = wv[lane]
                    i = g * 16 + lane
                    for j in range(d // 16):
                        sl = pl.ds(j * 16, 16)
                        rows_v[i, sl] = rows_v[i, sl] * w
                return 0
            lax.fori_loop(0, CH // 16, grp, 0)

            pltpu.sync_copy(rows_v, acc.at[dst_v], add=True)
            return 0
        lax.fori_loop(0, nchunk, chunk, 0)

        plsc.subcore_barrier()
        row0 = s * rpt
        pltpu.sync_copy(acc.at[pl.ds(row0, rpt)],
                        out_hbm.at[c, pl.ds(row0, rpt)])

    return body(h, src, dst, ew)


def kernel(x, edge_index, edge_weight, W1, b1, W2, b2, Wl, bl):
    src = edge_index[0].astype(jnp.int32)
    dst = edge_index[1].astype(jnp.int32)
    ew = edge_weight.astype(jnp.float32)

    e = src.shape[0]
    epad = ((e + NW * CH - 1) // (NW * CH)) * (NW * CH)
    padn = epad - e
    if padn:
        src = jnp.concatenate([src, jnp.zeros((padn,), jnp.int32)])
        dst = jnp.concatenate([dst, jnp.zeros((padn,), jnp.int32)])
        ew = jnp.concatenate([ew, jnp.zeros((padn,), jnp.float32)])

    n = x.shape[0]
    d2 = W2.shape[1]
    w2_pad = jnp.pad(W2, ((0, 0), (0, 128 - d2)))
    b2_pad = jnp.pad(b2, (0, 128 - d2))
    m_out = Wl.shape[1]
    wl_pad = jnp.pad(Wl, ((0, 128 - d2), (0, 128 - m_out)))
    bl_pad = jnp.pad(bl, (0, 128 - m_out))

    h1 = _tc_matmul(x, W1)
    part1 = _sc_scatter(h1, src, dst, ew)
    h2 = _tc_fused(part1[0, :n], part1[1, :n], b1, w2_pad,
                   jnp.zeros((128,), jnp.float32))
    part2 = _sc_scatter(h2, src, dst, ew)
    out = _tc_fused(part2[0, :n], part2[1, :n], b2_pad, wl_pad, bl_pad)
    return out[:, :m_out]
